# baseline (speedup 1.0000x reference)
import functools

import jax
import jax.numpy as jnp
from jax import lax
from jax.experimental import pallas as pl
from jax.experimental.pallas import tpu as pltpu
from jax.experimental.pallas import tpu_sc as plsc

B, S, F = 1024, 512, 128
NW = 32
B_SC = 832
B_TC = B - B_SC
EPW = B_SC // NW
NV = F // 16
NEG = -1e30
CHUNKS = ((0, 176), (176, 176), (352, 160))

_mesh = plsc.VectorSubcoreMesh(
    core_axis_name="c", subcore_axis_name="s", num_cores=2, num_subcores=16
)


def _row_loop(buf, lo, hi, acc):
    def body(r, acc):
        return tuple(
            jnp.maximum(acc[v], buf[r, pl.ds(v * 16, 16)]) for v in range(NV)
        )

    return lax.fori_loop(lo, hi, body, acc)


@functools.partial(
    pl.kernel,
    out_type=jax.ShapeDtypeStruct((NW, EPW, 3 * F), jnp.float32),
    mesh=_mesh,
    scratch_types=[
        pltpu.VMEM((3, 176, F), jnp.float32),
        pltpu.VMEM((EPW, 3 * F), jnp.float32),
        pltpu.VMEM((EPW, 16), jnp.int32),
        pltpu.SemaphoreType.DMA,
        pltpu.SemaphoreType.DMA,
        pltpu.SemaphoreType.DMA,
    ],
)
def _sc_piecewise_max(conv_hbm, ee_hbm, out_hbm, buf, out_v, e_v, sem0, sem1, sem2):
    wid = lax.axis_index("c") * 16 + lax.axis_index("s")
    base = wid * EPW

    pltpu.sync_copy(ee_hbm.at[wid], e_v)

    sems = (sem0, sem1, sem2)

    def dma(ex, q):
        c0, rows = CHUNKS[q]
        return pltpu.make_async_copy(
            conv_hbm.at[base + ex, pl.ds(c0, rows)],
            buf.at[q, pl.ds(0, rows)],
            sems[q],
        )

    dma(0, 0).start()
    dma(0, 1).start()

    def ex_body(i, carry):
        evec = e_v[i]
        e1s = evec[0]
        e2s = evec[1]
        neg = jnp.full((16,), NEG, jnp.float32)
        accs = [tuple(neg for _ in range(NV)) for _ in range(3)]
        for q in range(3):
            c0, rows = CHUNKS[q]
            dma(i, q).wait()
            if q == 0:
                dma(i, 2).start()
            else:

                @pl.when(i + 1 < EPW)
                def _():
                    dma(i + 1, q - 1).start()

            cbuf = buf.at[q]
            a = jnp.clip(e1s + 1 - c0, 0, rows)
            b = jnp.clip(e2s + 1 - c0, 0, rows)
            accs[0] = _row_loop(cbuf, 0, a, accs[0])
            accs[1] = _row_loop(cbuf, a, b, accs[1])
            accs[2] = _row_loop(cbuf, b, rows, accs[2])

        for p in range(3):
            for v in range(NV):
                out_v[i, pl.ds(p * F + v * 16, 16)] = accs[p][v]
        return carry

    lax.fori_loop(0, EPW, ex_body, 0)
    pltpu.sync_copy(out_v, out_hbm.at[wid])


EX_BLK = 8


def _tc_body(e1_ref, e2_ref, x_ref, out_ref):
    pos = lax.broadcasted_iota(jnp.int32, (S, F), 0)
    neg = jnp.float32(NEG)
    for j in range(EX_BLK):
        e1s = e1_ref[j, 0, 0]
        e2s = e2_ref[j, 0, 0]
        x = x_ref[j]
        out_ref[j, 0, pl.ds(0, F)] = jnp.max(jnp.where(pos <= e1s, x, neg), axis=0)
        out_ref[j, 0, pl.ds(F, F)] = jnp.max(
            jnp.where((pos > e1s) & (pos <= e2s), x, neg), axis=0
        )
        out_ref[j, 0, pl.ds(2 * F, F)] = jnp.max(
            jnp.where(pos > e2s, x, neg), axis=0
        )


_tc_piecewise_max = pl.pallas_call(
    _tc_body,
    grid=(B_TC // EX_BLK,),
    in_specs=[
        pl.BlockSpec(
            (EX_BLK, 1, 1),
            lambda i: (i + B_SC // EX_BLK, 0, 0),
            memory_space=pltpu.SMEM,
        ),
        pl.BlockSpec(
            (EX_BLK, 1, 1),
            lambda i: (i + B_SC // EX_BLK, 0, 0),
            memory_space=pltpu.SMEM,
        ),
        pl.BlockSpec((EX_BLK, S, F), lambda i: (i + B_SC // EX_BLK, 0, 0)),
    ],
    out_specs=pl.BlockSpec((EX_BLK, 1, 3 * F), lambda i: (i, 0, 0)),
    out_shape=jax.ShapeDtypeStruct((B_TC, 1, 3 * F), jnp.float32),
)


def kernel(conv_output, e1, e2):
    e1i = e1.astype(jnp.int32)
    e2i = e2.astype(jnp.int32)
    ee = jnp.pad(jnp.concatenate([e1i, e2i], axis=1), ((0, 0), (0, 14)))
    ee3 = ee[:B_SC].reshape(NW, EPW, 16)
    sc_out = _sc_piecewise_max(conv_output, ee3).reshape(B_SC, 3 * F)
    tc_out = _tc_piecewise_max(
        e1i.reshape(B, 1, 1), e2i.reshape(B, 1, 1), conv_output
    ).reshape(B_TC, 3 * F)
    return jnp.concatenate([sc_out, tc_out], axis=0)


# speedup vs baseline: 1.0099x; 1.0099x over previous
import functools

import jax
import jax.numpy as jnp
from jax import lax
from jax.experimental import pallas as pl
from jax.experimental.pallas import tpu as pltpu
from jax.experimental.pallas import tpu_sc as plsc

B, S, F = 1024, 512, 128
NW = 32
B_SC = 768
B_TC = B - B_SC
EPW = B_SC // NW
NV = F // 16
NEG = -1e30
CHUNKS = ((0, 176), (176, 176), (352, 160))

_mesh = plsc.VectorSubcoreMesh(
    core_axis_name="c", subcore_axis_name="s", num_cores=2, num_subcores=16
)


def _row_loop(buf, lo, hi, acc):
    def body(r, acc):
        return tuple(
            jnp.maximum(acc[v], buf[r, pl.ds(v * 16, 16)]) for v in range(NV)
        )

    return lax.fori_loop(lo, hi, body, acc)


@functools.partial(
    pl.kernel,
    out_type=jax.ShapeDtypeStruct((NW, EPW, 3 * F), jnp.float32),
    mesh=_mesh,
    scratch_types=[
        pltpu.VMEM((3, 176, F), jnp.float32),
        pltpu.VMEM((EPW, 3 * F), jnp.float32),
        pltpu.VMEM((EPW, 16), jnp.int32),
        pltpu.SemaphoreType.DMA,
        pltpu.SemaphoreType.DMA,
        pltpu.SemaphoreType.DMA,
    ],
)
def _sc_piecewise_max(conv_hbm, ee_hbm, out_hbm, buf, out_v, e_v, sem0, sem1, sem2):
    wid = lax.axis_index("c") * 16 + lax.axis_index("s")
    base = wid * EPW

    pltpu.sync_copy(ee_hbm.at[wid], e_v)

    sems = (sem0, sem1, sem2)

    def dma(ex, q):
        c0, rows = CHUNKS[q]
        return pltpu.make_async_copy(
            conv_hbm.at[base + ex, pl.ds(c0, rows)],
            buf.at[q, pl.ds(0, rows)],
            sems[q],
        )

    dma(0, 0).start()
    dma(0, 1).start()

    def ex_body(i, carry):
        evec = e_v[i]
        e1s = evec[0]
        e2s = evec[1]
        neg = jnp.full((16,), NEG, jnp.float32)
        accs = [tuple(neg for _ in range(NV)) for _ in range(3)]
        for q in range(3):
            c0, rows = CHUNKS[q]
            dma(i, q).wait()
            if q == 0:
                dma(i, 2).start()
            else:

                @pl.when(i + 1 < EPW)
                def _():
                    dma(i + 1, q - 1).start()

            cbuf = buf.at[q]
            a = jnp.clip(e1s + 1 - c0, 0, rows)
            b = jnp.clip(e2s + 1 - c0, 0, rows)
            accs[0] = _row_loop(cbuf, 0, a, accs[0])
            accs[1] = _row_loop(cbuf, a, b, accs[1])
            accs[2] = _row_loop(cbuf, b, rows, accs[2])

        for p in range(3):
            for v in range(NV):
                out_v[i, pl.ds(p * F + v * 16, 16)] = accs[p][v]
        return carry

    lax.fori_loop(0, EPW, ex_body, 0)
    pltpu.sync_copy(out_v, out_hbm.at[wid])


EX_BLK = 8


def _tc_body(e1_ref, e2_ref, x_ref, out_ref):
    G = S // 8
    gi = lax.broadcasted_iota(jnp.int32, (G, F), 0)
    pos8 = lax.broadcasted_iota(jnp.int32, (8, F), 0)
    neg = jnp.float32(NEG)
    for j in range(EX_BLK):
        e1s = e1_ref[j, 0, 0]
        e2s = e2_ref[j, 0, 0]
        x = x_ref[j]
        g = jnp.max(x.reshape(G, 8, F), axis=1)
        ge1 = e1s // 8
        ge2 = e2s // 8
        q1 = (e1s + 1) // 8
        q2hi = (e2s + 1) // 8
        p1 = jnp.max(jnp.where(gi < q1, g, neg), axis=0)
        p2 = jnp.max(jnp.where((gi > ge1) & (gi < q2hi), g, neg), axis=0)
        p3 = jnp.max(jnp.where(gi > ge2, g, neg), axis=0)
        xe1 = x_ref[j, pl.ds(ge1 * 8, 8), :]
        re1 = pos8 + ge1 * 8
        p1 = jnp.maximum(p1, jnp.max(jnp.where(re1 <= e1s, xe1, neg), axis=0))
        p2 = jnp.maximum(
            p2, jnp.max(jnp.where((re1 > e1s) & (re1 <= e2s), xe1, neg), axis=0)
        )
        xe2 = x_ref[j, pl.ds(ge2 * 8, 8), :]
        re2 = pos8 + ge2 * 8
        p2 = jnp.maximum(
            p2, jnp.max(jnp.where((re2 > e1s) & (re2 <= e2s), xe2, neg), axis=0)
        )
        p3 = jnp.maximum(p3, jnp.max(jnp.where(re2 > e2s, xe2, neg), axis=0))
        out_ref[j, 0, pl.ds(0, F)] = p1
        out_ref[j, 0, pl.ds(F, F)] = p2
        out_ref[j, 0, pl.ds(2 * F, F)] = p3


_tc_piecewise_max = pl.pallas_call(
    _tc_body,
    grid=(B_TC // EX_BLK,),
    in_specs=[
        pl.BlockSpec(
            (EX_BLK, 1, 1),
            lambda i: (i + B_SC // EX_BLK, 0, 0),
            memory_space=pltpu.SMEM,
        ),
        pl.BlockSpec(
            (EX_BLK, 1, 1),
            lambda i: (i + B_SC // EX_BLK, 0, 0),
            memory_space=pltpu.SMEM,
        ),
        pl.BlockSpec((EX_BLK, S, F), lambda i: (i + B_SC // EX_BLK, 0, 0)),
    ],
    out_specs=pl.BlockSpec((EX_BLK, 1, 3 * F), lambda i: (i, 0, 0)),
    out_shape=jax.ShapeDtypeStruct((B_TC, 1, 3 * F), jnp.float32),
)


def kernel(conv_output, e1, e2):
    e1i = e1.astype(jnp.int32)
    e2i = e2.astype(jnp.int32)
    ee = jnp.pad(jnp.concatenate([e1i, e2i], axis=1), ((0, 0), (0, 14)))
    ee3 = ee[:B_SC].reshape(NW, EPW, 16)
    sc_out = _sc_piecewise_max(conv_output, ee3).reshape(B_SC, 3 * F)
    tc_out = _tc_piecewise_max(
        e1i.reshape(B, 1, 1), e2i.reshape(B, 1, 1), conv_output
    ).reshape(B_TC, 3 * F)
    return jnp.concatenate([sc_out, tc_out], axis=0)


# speedup vs baseline: 1.0208x; 1.0108x over previous
import functools

import jax
import jax.numpy as jnp
from jax import lax
from jax.experimental import pallas as pl
from jax.experimental.pallas import tpu as pltpu
from jax.experimental.pallas import tpu_sc as plsc

B, S, F = 1024, 512, 128
NW = 32
B_SC = 640
B_TC = B - B_SC
EPW = B_SC // NW
NV = F // 16
NEG = -1e30
CHUNKS = ((0, 176), (176, 176), (352, 160))

_mesh = plsc.VectorSubcoreMesh(
    core_axis_name="c", subcore_axis_name="s", num_cores=2, num_subcores=16
)


def _row_loop(buf, lo, hi, acc):
    def body(r, acc):
        return tuple(
            jnp.maximum(acc[v], buf[r, pl.ds(v * 16, 16)]) for v in range(NV)
        )

    return lax.fori_loop(lo, hi, body, acc)


@functools.partial(
    pl.kernel,
    out_type=jax.ShapeDtypeStruct((NW, EPW, 3 * F), jnp.float32),
    mesh=_mesh,
    scratch_types=[
        pltpu.VMEM((3, 176, F), jnp.float32),
        pltpu.VMEM((EPW, 3 * F), jnp.float32),
        pltpu.VMEM((EPW, 16), jnp.int32),
        pltpu.SemaphoreType.DMA,
        pltpu.SemaphoreType.DMA,
        pltpu.SemaphoreType.DMA,
    ],
)
def _sc_piecewise_max(conv_hbm, ee_hbm, out_hbm, buf, out_v, e_v, sem0, sem1, sem2):
    wid = lax.axis_index("c") * 16 + lax.axis_index("s")
    base = wid * EPW

    pltpu.sync_copy(ee_hbm.at[wid], e_v)

    sems = (sem0, sem1, sem2)

    def dma(ex, q):
        c0, rows = CHUNKS[q]
        return pltpu.make_async_copy(
            conv_hbm.at[base + ex, pl.ds(c0, rows)],
            buf.at[q, pl.ds(0, rows)],
            sems[q],
        )

    dma(0, 0).start()
    dma(0, 1).start()

    def ex_body(i, carry):
        evec = e_v[i]
        e1s = evec[0]
        e2s = evec[1]
        neg = jnp.full((16,), NEG, jnp.float32)
        accs = [tuple(neg for _ in range(NV)) for _ in range(3)]
        for q in range(3):
            c0, rows = CHUNKS[q]
            dma(i, q).wait()
            if q == 0:
                dma(i, 2).start()
            else:

                @pl.when(i + 1 < EPW)
                def _():
                    dma(i + 1, q - 1).start()

            cbuf = buf.at[q]
            a = jnp.clip(e1s + 1 - c0, 0, rows)
            b = jnp.clip(e2s + 1 - c0, 0, rows)
            accs[0] = _row_loop(cbuf, 0, a, accs[0])
            accs[1] = _row_loop(cbuf, a, b, accs[1])
            accs[2] = _row_loop(cbuf, b, rows, accs[2])

        for p in range(3):
            for v in range(NV):
                out_v[i, pl.ds(p * F + v * 16, 16)] = accs[p][v]
        return carry

    lax.fori_loop(0, EPW, ex_body, 0)
    pltpu.sync_copy(out_v, out_hbm.at[wid])


EX_BLK = 8


def _tc_body(e1_ref, e2_ref, x_ref, out_ref):
    G = S // 8
    gi = lax.broadcasted_iota(jnp.int32, (G, F), 0)
    pos8 = lax.broadcasted_iota(jnp.int32, (8, F), 0)
    neg = jnp.float32(NEG)
    for j in range(EX_BLK):
        e1s = e1_ref[j, 0, 0]
        e2s = e2_ref[j, 0, 0]
        x = x_ref[j]
        g = jnp.max(x.reshape(G, 8, F), axis=1)
        ge1 = e1s // 8
        ge2 = e2s // 8
        q1 = (e1s + 1) // 8
        q2hi = (e2s + 1) // 8
        p1 = jnp.max(jnp.where(gi < q1, g, neg), axis=0)
        p2 = jnp.max(jnp.where((gi > ge1) & (gi < q2hi), g, neg), axis=0)
        p3 = jnp.max(jnp.where(gi > ge2, g, neg), axis=0)
        xe1 = x_ref[j, pl.ds(ge1 * 8, 8), :]
        re1 = pos8 + ge1 * 8
        p1 = jnp.maximum(p1, jnp.max(jnp.where(re1 <= e1s, xe1, neg), axis=0))
        p2 = jnp.maximum(
            p2, jnp.max(jnp.where((re1 > e1s) & (re1 <= e2s), xe1, neg), axis=0)
        )
        xe2 = x_ref[j, pl.ds(ge2 * 8, 8), :]
        re2 = pos8 + ge2 * 8
        p2 = jnp.maximum(
            p2, jnp.max(jnp.where((re2 > e1s) & (re2 <= e2s), xe2, neg), axis=0)
        )
        p3 = jnp.maximum(p3, jnp.max(jnp.where(re2 > e2s, xe2, neg), axis=0))
        out_ref[j, 0, pl.ds(0, F)] = p1
        out_ref[j, 0, pl.ds(F, F)] = p2
        out_ref[j, 0, pl.ds(2 * F, F)] = p3


_tc_piecewise_max = pl.pallas_call(
    _tc_body,
    grid=(B_TC // EX_BLK,),
    in_specs=[
        pl.BlockSpec(
            (EX_BLK, 1, 1),
            lambda i: (i + B_SC // EX_BLK, 0, 0),
            memory_space=pltpu.SMEM,
        ),
        pl.BlockSpec(
            (EX_BLK, 1, 1),
            lambda i: (i + B_SC // EX_BLK, 0, 0),
            memory_space=pltpu.SMEM,
        ),
        pl.BlockSpec((EX_BLK, S, F), lambda i: (i + B_SC // EX_BLK, 0, 0)),
    ],
    out_specs=pl.BlockSpec((EX_BLK, 1, 3 * F), lambda i: (i, 0, 0)),
    out_shape=jax.ShapeDtypeStruct((B_TC, 1, 3 * F), jnp.float32),
)


def kernel(conv_output, e1, e2):
    e1i = e1.astype(jnp.int32)
    e2i = e2.astype(jnp.int32)
    ee = jnp.pad(jnp.concatenate([e1i, e2i], axis=1), ((0, 0), (0, 14)))
    ee3 = ee[:B_SC].reshape(NW, EPW, 16)
    sc_out = _sc_piecewise_max(conv_output, ee3).reshape(B_SC, 3 * F)
    tc_out = _tc_piecewise_max(
        e1i.reshape(B, 1, 1), e2i.reshape(B, 1, 1), conv_output
    ).reshape(B_TC, 3 * F)
    return jnp.concatenate([sc_out, tc_out], axis=0)
